# Initial kernel scaffold; baseline (speedup 1.0000x reference)
import jax
import jax.numpy as jnp
from jax.experimental import pallas as pl


def kernel(pred_logits, pred_boxes, target_sizes):
    raise NotImplementedError("write your pallas kernel here")


# speedup vs baseline: 1.0297x; 1.0297x over previous
import jax
import jax.numpy as jnp
from jax.experimental import pallas as pl


def _convert_scale_kernel(a_ref, b_ref, m_ref, s_ref, o_ref):
    a = a_ref[...]
    b = b_ref[...]
    m = m_ref[...]
    s = s_ref[...]
    lo = a - 0.5 * b
    hi = b + 0.5 * a
    o_ref[...] = (m * lo + (1.0 - m) * hi) * s


def kernel(pred_logits, pred_boxes, target_sizes):
    bs, nq, nc = pred_logits.shape
    t = pred_boxes.shape[2]
    prob = jax.nn.sigmoid(pred_logits)
    topk_values, topk_indexes = jax.lax.top_k(prob.reshape(bs, -1), 100)
    scores = topk_values
    topk_boxes = topk_indexes // nc
    labels = topk_indexes % nc
    g = jnp.take_along_axis(
        pred_boxes, topk_boxes[:, :, None, None], axis=1
    )
    a = g.reshape(bs, 100 * t * 4)
    b = g.reshape(bs, 100 * t, 2, 2)[:, :, ::-1, :].reshape(bs, 100 * t * 4)
    lane = jnp.arange(100 * t * 4) % 4
    m = (lane < 2).astype(jnp.float32)[None, :] * jnp.ones((bs, 1), jnp.float32)
    img_h = target_sizes[:, 0].astype(jnp.float32)
    img_w = target_sizes[:, 1].astype(jnp.float32)
    scale4 = jnp.stack([img_w, img_h, img_w, img_h], axis=1)
    s = jnp.tile(scale4, (1, 100 * t))
    out = pl.pallas_call(
        _convert_scale_kernel,
        out_shape=jax.ShapeDtypeStruct((bs, 100 * t * 4), jnp.float32),
    )(a, b, m, s)
    boxes = out.reshape(bs, 100, t, 4)
    return scores, labels, boxes


# speedup vs baseline: 1.6096x; 1.5632x over previous
import jax
import jax.numpy as jnp
from jax import lax
from jax.experimental import pallas as pl
from jax.experimental.pallas import tpu as pltpu
from jax.experimental.pallas import tpu_sc as plsc

BS, NQ, NC, T = 16, 1000, 365, 36
N = NQ * NC
NPAD = 368640
R = NPAD // 128
K = 100
D = 10
W = D * 128
BROW = 256


def _topk_kernel(x_ref, ck_ref, ci_ref, msk_ref):
    x = x_ref[0]
    u = lax.bitcast_convert_type(x, jnp.uint32)
    neg = (u >> jnp.uint32(31)) == jnp.uint32(1)
    up = jnp.where(neg, ~u, u | jnp.uint32(0x80000000))

    def body(i, tau):
        t = tau | (jnp.uint32(1) << jnp.uint32(31 - i))
        cnt = jnp.sum((up >= t).astype(jnp.int32))
        return jnp.where(cnt >= K, t, tau)

    tau = lax.fori_loop(0, 32, body, jnp.uint32(0))

    upi = lax.bitcast_convert_type(up, jnp.int32)
    rows = lax.broadcasted_iota(jnp.int32, (R, 128), 0)
    gidx = rows * 128 + lax.broadcasted_iota(jnp.int32, (R, 128), 1)

    msk_ref[...] = (up >= tau).astype(jnp.int32)

    def ebody(d, carry):
        mask = msk_ref[...] == 1
        first_r = jnp.min(jnp.where(mask, rows, R), axis=0)
        sel = rows == first_r[None, :]
        hit = sel & mask
        ck_ref[0, pl.ds(d, 1)] = jnp.sum(jnp.where(hit, upi, 0),
                                         axis=0)[None, :]
        ci_ref[0, pl.ds(d, 1)] = jnp.sum(jnp.where(hit, gidx, 0),
                                         axis=0)[None, :]
        msk_ref[...] = (mask & ~sel).astype(jnp.int32)
        return carry

    lax.fori_loop(0, D, ebody, jnp.int32(0))


def _rank_kernel(ck_ref, ci_ref, sc_ref, lb_ref, gq_ref):
    b = pl.program_id(0)

    def rbody(c, rank):
        kc = lax.bitcast_convert_type(ck_ref[0, pl.ds(c, 1), :],
                                      jnp.uint32)[0]
        ic = ci_ref[0, pl.ds(c, 1), :][0]
        rows = []
        for d in range(D):
            kd = lax.bitcast_convert_type(ck_ref[0, d], jnp.uint32)
            idd = ci_ref[0, d]
            beats = (kc[None, :] > kd[:, None]) | (
                (kc[None, :] == kd[:, None]) & (ic[None, :] < idd[:, None]))
            rows.append(jnp.sum(beats.astype(jnp.int32), axis=1))
        return rank + jnp.stack(rows, axis=0)

    rank = lax.fori_loop(0, D, rbody, jnp.zeros((D, 128), jnp.int32))

    s = lax.iota(jnp.int32, 128)[:, None]
    sel_i = jnp.zeros((128,), jnp.int32)
    sel_hi = jnp.zeros((128,), jnp.int32)
    sel_lo = jnp.zeros((128,), jnp.int32)
    for d in range(D):
        oh = (rank[d][None, :] == s).astype(jnp.int32)
        kd = lax.bitcast_convert_type(ck_ref[0, d], jnp.uint32)
        idd = ci_ref[0, d]
        khi = (kd >> jnp.uint32(16)).astype(jnp.int32)
        klo = (kd & jnp.uint32(0xFFFF)).astype(jnp.int32)
        sel_i = sel_i + jnp.sum(oh * idd[None, :], axis=1)
        sel_hi = sel_hi + jnp.sum(oh * khi[None, :], axis=1)
        sel_lo = sel_lo + jnp.sum(oh * klo[None, :], axis=1)

    key = (sel_hi.astype(jnp.uint32) << jnp.uint32(16)) | sel_lo.astype(
        jnp.uint32)
    pos = (key >> jnp.uint32(31)) == jnp.uint32(1)
    fbits = jnp.where(pos, key ^ jnp.uint32(0x80000000), ~key)
    f = lax.bitcast_convert_type(fbits, jnp.float32)
    sc_ref[0, 0] = 1.0 / (1.0 + jnp.exp(-f))
    lb_ref[0, 0] = sel_i % NC
    gq = b * NQ + sel_i // NC
    gq_ref[0, 0] = jnp.clip(gq, 0, BS * NQ - 1)


def _gather_body(boxes_hbm, boxsw_hbm, qidx_hbm, scale_hbm, out_hbm, idx_v,
                 rows_v, rsw_v, scl_v, sem, sem2):
    wid = lax.axis_index("s") * 2 + lax.axis_index("c")
    pltpu.sync_copy(qidx_hbm.at[pl.ds(wid * 56, 56)], idx_v)
    pltpu.sync_copy(scale_hbm.at[wid, 0], scl_v)
    pltpu.async_copy(boxes_hbm.at[idx_v], rows_v, sem).wait()
    pltpu.async_copy(boxsw_hbm.at[idx_v], rsw_v, sem2).wait()
    sv = scl_v[...]
    lane = lax.iota(jnp.int32, 16)
    mhi = ((lane >> 1) & 1).astype(jnp.float32)
    mlo = 1.0 - mhi

    def body(r, carry):
        for j in range(9):
            v = rows_v[r, pl.ds(j * 16, 16)]
            vs = rsw_v[r, pl.ds(j * 16, 16)]
            res = ((v - 0.5 * vs) * mlo + (vs + 0.5 * v) * mhi) * sv
            rows_v[r, pl.ds(j * 16, 16)] = res
        return carry

    lax.fori_loop(0, 56, body, jnp.int32(0))
    pltpu.sync_copy(rows_v, out_hbm.at[wid])


def kernel(pred_logits, pred_boxes, target_sizes):
    bs, nq, nc = pred_logits.shape
    t = pred_boxes.shape[2]

    x = pred_logits.reshape(bs, N)
    x = jnp.pad(x, ((0, 0), (0, NPAD - N)), constant_values=-jnp.inf)
    x = x.reshape(bs, R, 128)

    ck, ci = pl.pallas_call(
        _topk_kernel,
        grid=(bs,),
        in_specs=[pl.BlockSpec((1, R, 128), lambda b: (b, 0, 0))],
        out_specs=[
            pl.BlockSpec((1, D, 128), lambda b: (b, 0, 0)),
            pl.BlockSpec((1, D, 128), lambda b: (b, 0, 0)),
        ],
        out_shape=[
            jax.ShapeDtypeStruct((bs, D, 128), jnp.int32),
            jax.ShapeDtypeStruct((bs, D, 128), jnp.int32),
        ],
        scratch_shapes=[pltpu.VMEM((R, 128), jnp.int32)],
    )(x)

    scores128, labels128, gq128 = pl.pallas_call(
        _rank_kernel,
        grid=(bs,),
        in_specs=[
            pl.BlockSpec((1, D, 128), lambda b: (b, 0, 0)),
            pl.BlockSpec((1, D, 128), lambda b: (b, 0, 0)),
        ],
        out_specs=[
            pl.BlockSpec((1, 1, 128), lambda b: (b, 0, 0)),
            pl.BlockSpec((1, 1, 128), lambda b: (b, 0, 0)),
            pl.BlockSpec((1, 1, 128), lambda b: (b, 0, 0)),
        ],
        out_shape=[
            jax.ShapeDtypeStruct((bs, 1, 128), jnp.float32),
            jax.ShapeDtypeStruct((bs, 1, 128), jnp.int32),
            jax.ShapeDtypeStruct((bs, 1, 128), jnp.int32),
        ],
    )(ck, ci)

    scores = scores128[:, 0, :K]
    labels = labels128[:, 0, :K]
    gq = gq128[:, 0, :K]

    gq_pad = jnp.pad(gq.reshape(bs, 2, 50), ((0, 0), (0, 0), (0, 6)))
    qidx = gq_pad.reshape(bs * 2 * 56)

    img_h = target_sizes[:, 0].astype(jnp.float32)
    img_w = target_sizes[:, 1].astype(jnp.float32)
    scale4 = jnp.stack([img_w, img_h, img_w, img_h], axis=1)
    scale16 = jnp.tile(scale4, (1, 4))
    scale32 = jnp.repeat(scale16, 2, axis=0).reshape(2 * bs, 1, 16)

    boxes_flat = pred_boxes.reshape(bs * nq, t * 4)
    boxes_flat = jnp.pad(boxes_flat, ((0, 0), (0, BROW - t * 4)))
    boxes_sw = pred_boxes[..., [2, 3, 0, 1]].reshape(bs * nq, t * 4)
    boxes_sw = jnp.pad(boxes_sw, ((0, 0), (0, BROW - t * 4)))

    mesh = plsc.VectorSubcoreMesh(core_axis_name="c", subcore_axis_name="s")
    boxes_out = pl.kernel(
        _gather_body,
        out_type=jax.ShapeDtypeStruct((2 * bs, 56, BROW), jnp.float32),
        mesh=mesh,
        scratch_types=[
            pltpu.VMEM((56,), jnp.int32),
            pltpu.VMEM((56, BROW), jnp.float32),
            pltpu.VMEM((56, BROW), jnp.float32),
            pltpu.VMEM((16,), jnp.float32),
            pltpu.SemaphoreType.DMA,
            pltpu.SemaphoreType.DMA,
        ],
    )(boxes_flat, boxes_sw, qidx, scale32)

    boxes = boxes_out[:, :50, :t * 4].reshape(bs, K, t, 4)
    return scores, labels, boxes
